# baseline (speedup 1.0000x reference)
import functools

import jax
import jax.numpy as jnp
from jax import lax
from jax.experimental import pallas as pl
from jax.experimental.pallas import tpu as pltpu
from jax.experimental.pallas import tpu_sc as plsc

B = 16
D = 128
TOTAL = 32768
NW = 32
RPW = TOTAL // NW
CH = 256
NCH = RPW // CH
LANES = 16
G = D // LANES


def _sc_partial_sums(x2d, nv):
    mesh = plsc.VectorSubcoreMesh(core_axis_name="c", subcore_axis_name="s")

    @functools.partial(
        pl.kernel,
        mesh=mesh,
        out_type=jax.ShapeDtypeStruct((NW * B * D,), jnp.float32),
        scratch_types=[
            pltpu.VMEM((B,), jnp.int32),
            pltpu.VMEM((CH, D), jnp.float32),
            pltpu.VMEM((CH, D), jnp.float32),
            pltpu.VMEM((CH, D), jnp.float32),
            pltpu.VMEM((B * D,), jnp.float32),
            pltpu.SMEM((B + 1,), jnp.int32),
            pltpu.SemaphoreType.DMA,
            pltpu.SemaphoreType.DMA,
            pltpu.SemaphoreType.DMA,
        ],
        compiler_params=pltpu.CompilerParams(needs_layout_passes=False),
    )
    def k(x_hbm, nv_hbm, part_hbm, nv_v, b0, b1, b2, acc_v, bnd_s,
          s0, s1, s2):
        wid = lax.axis_index("s") * 2 + lax.axis_index("c")

        pltpu.sync_copy(nv_hbm, nv_v)
        nv_vec = nv_v[...]
        lanes = lax.iota(jnp.int32, LANES)

        incl = nv_vec
        for sh in (1, 2, 4, 8):
            shifted = incl.at[jnp.maximum(lanes - sh, 0)].get(
                mode="promise_in_bounds")
            incl = incl + jnp.where(lanes >= sh, shifted, 0)
        starts_vec = incl - nv_vec
        for s in range(B):
            bnd_s[s] = jnp.sum(jnp.where(lanes == s, starts_vec, 0))
        bnd_s[B] = jnp.sum(jnp.where(lanes == B - 1, incl, 0))

        zero = jnp.zeros((LANES,), jnp.float32)
        for i in range(B * D // LANES):
            acc_v[pl.ds(i * LANES, LANES)] = zero

        row_lo = wid * RPW
        bufs = [b0, b1, b2]
        sems = [s0, s1, s2]

        def start_chunk(kk, which):
            base = row_lo + kk * CH
            return pltpu.async_copy(
                x_hbm.at[pl.ds(base, CH), :], bufs[which], sems[which])

        def consume(kk, buf_v):
            base = row_lo + kk * CH

            def seg_body(s, carry):
                a = jnp.clip(bnd_s[s] - base, 0, CH)
                b = jnp.clip(bnd_s[s + 1] - base, 0, CH)

                @pl.when(b > a)
                def _():
                    zcarry = tuple(zero for _ in range(G))

                    @plsc.parallel_loop(a, b, unroll=2, carry=zcarry)
                    def row_body(rr, vs):
                        return tuple(
                            vs[g] + buf_v[rr, pl.ds(g * LANES, LANES)]
                            for g in range(G))

                    for g in range(G):
                        off = s * D + g * LANES
                        acc_v[pl.ds(off, LANES)] += row_body[g]

                return carry

            lax.fori_loop(0, B, seg_body, 0)

        NBUF = 3
        for kk in range(NBUF - 1):
            start_chunk(kk, kk)
        for kk in range(NCH):
            w = kk % NBUF
            pltpu.make_async_copy(
                x_hbm.at[pl.ds(0, CH), :], bufs[w], sems[w]).wait()
            if kk + NBUF - 1 < NCH:
                start_chunk(kk + NBUF - 1, (kk + NBUF - 1) % NBUF)
            consume(kk, bufs[w])
        pltpu.sync_copy(acc_v, part_hbm.at[pl.ds(wid * B * D, B * D)])

    return k(x2d, nv)


def _tc_combine(partials, nv):
    def body(p_ref, c_ref, o_ref):
        acc = p_ref[0]
        for i in range(1, NW):
            acc = acc + p_ref[i]
        counts = jnp.maximum(c_ref[...].astype(jnp.float32), 1.0)
        o_ref[...] = acc / counts

    return pl.pallas_call(
        body,
        out_shape=jax.ShapeDtypeStruct((B, D), jnp.float32),
    )(partials, nv)


def kernel(inputs, nv_in):
    part = _sc_partial_sums(inputs, nv_in)
    partials = part.reshape(NW, B, D)
    return _tc_combine(partials, nv_in.reshape(B, 1))


# speedup vs baseline: 1.0023x; 1.0023x over previous
import functools

import jax
import jax.numpy as jnp
from jax import lax
from jax.experimental import pallas as pl
from jax.experimental.pallas import tpu as pltpu
from jax.experimental.pallas import tpu_sc as plsc

B = 16
D = 128
TOTAL = 32768
NW = 32
RPW = TOTAL // NW
CH = 256
NCH = RPW // CH
LANES = 16
G = D // LANES


def _sc_partial_sums(x_flat, nv):
    mesh = plsc.VectorSubcoreMesh(core_axis_name="c", subcore_axis_name="s")

    @functools.partial(
        pl.kernel,
        mesh=mesh,
        out_type=jax.ShapeDtypeStruct((NW * B * D,), jnp.float32),
        scratch_types=[
            pltpu.VMEM((B,), jnp.int32),
            pltpu.VMEM((CH * D,), jnp.float32),
            pltpu.VMEM((CH * D,), jnp.float32),
            pltpu.VMEM((CH * D,), jnp.float32),
            pltpu.VMEM((B * D,), jnp.float32),
            pltpu.SMEM((B + 1,), jnp.int32),
            pltpu.SemaphoreType.DMA,
            pltpu.SemaphoreType.DMA,
            pltpu.SemaphoreType.DMA,
        ],
        compiler_params=pltpu.CompilerParams(needs_layout_passes=False),
    )
    def k(x_hbm, nv_hbm, part_hbm, nv_v, b0, b1, b2, acc_v, bnd_s,
          s0, s1, s2):
        wid = lax.axis_index("s") * 2 + lax.axis_index("c")

        pltpu.sync_copy(nv_hbm, nv_v)
        nv_vec = nv_v[...]
        lanes = lax.iota(jnp.int32, LANES)

        incl = nv_vec
        for sh in (1, 2, 4, 8):
            shifted = incl.at[jnp.maximum(lanes - sh, 0)].get(
                mode="promise_in_bounds")
            incl = incl + jnp.where(lanes >= sh, shifted, 0)
        starts_vec = incl - nv_vec
        for s in range(B):
            bnd_s[s] = jnp.sum(jnp.where(lanes == s, starts_vec, 0))
        bnd_s[B] = jnp.sum(jnp.where(lanes == B - 1, incl, 0))

        zero = jnp.zeros((LANES,), jnp.float32)
        for i in range(B * D // LANES):
            acc_v[pl.ds(i * LANES, LANES)] = zero

        row_lo = wid * RPW
        bufs = [b0, b1, b2]
        sems = [s0, s1, s2]

        def start_chunk(kk, which):
            base = row_lo + kk * CH
            return pltpu.async_copy(
                x_hbm.at[pl.ds(base * D, CH * D)], bufs[which], sems[which])

        def consume(kk, buf_v):
            base = row_lo + kk * CH

            def seg_body(s, carry):
                a = jnp.clip(bnd_s[s] - base, 0, CH)
                b = jnp.clip(bnd_s[s + 1] - base, 0, CH)

                @pl.when(b > a)
                def _():
                    zcarry = tuple(zero for _ in range(G))

                    @plsc.parallel_loop(a, b, unroll=2, carry=zcarry)
                    def row_body(rr, vs):
                        return tuple(
                            vs[g] + buf_v[pl.ds(rr * D + g * LANES, LANES)]
                            for g in range(G))

                    for g in range(G):
                        off = s * D + g * LANES
                        acc_v[pl.ds(off, LANES)] += row_body[g]

                return carry

            lax.fori_loop(0, B, seg_body, 0)

        NBUF = 3
        for kk in range(NBUF - 1):
            start_chunk(kk, kk)
        for kk in range(NCH):
            w = kk % NBUF
            pltpu.make_async_copy(
                x_hbm.at[pl.ds(0, CH * D)], bufs[w], sems[w]).wait()
            if kk + NBUF - 1 < NCH:
                start_chunk(kk + NBUF - 1, (kk + NBUF - 1) % NBUF)
            consume(kk, bufs[w])
        pltpu.sync_copy(acc_v, part_hbm.at[pl.ds(wid * B * D, B * D)])

    return k(x_flat, nv)


def _tc_combine(partials, nv):
    def body(p_ref, c_ref, o_ref):
        acc = p_ref[0]
        for i in range(1, NW):
            acc = acc + p_ref[i]
        counts = jnp.maximum(c_ref[...].astype(jnp.float32), 1.0)
        o_ref[...] = acc / counts

    return pl.pallas_call(
        body,
        out_shape=jax.ShapeDtypeStruct((B, D), jnp.float32),
    )(partials, nv)


def kernel(inputs, nv_in):
    x_flat = inputs.reshape(-1)
    part = _sc_partial_sums(x_flat, nv_in)
    partials = part.reshape(NW, B, D)
    return _tc_combine(partials, nv_in.reshape(B, 1))


# speedup vs baseline: 1.0948x; 1.0923x over previous
import functools

import jax
import jax.numpy as jnp
from jax import lax
from jax.experimental import pallas as pl
from jax.experimental.pallas import tpu as pltpu
from jax.experimental.pallas import tpu_sc as plsc

B = 16
D = 128
TOTAL = 32768
SC_ROWS = TOTAL // 2
NW = 32
RPW = SC_ROWS // NW
CH = 128
NCH = RPW // CH
LANES = 16
G = D // LANES
RB = 2048


def _sc_partial_sums(x_flat, nv):
    mesh = plsc.VectorSubcoreMesh(core_axis_name="c", subcore_axis_name="s")

    @functools.partial(
        pl.kernel,
        mesh=mesh,
        out_type=jax.ShapeDtypeStruct((NW * B * D,), jnp.float32),
        scratch_types=[
            pltpu.VMEM((B,), jnp.int32),
            pltpu.VMEM((CH * D,), jnp.float32),
            pltpu.VMEM((CH * D,), jnp.float32),
            pltpu.VMEM((CH * D,), jnp.float32),
            pltpu.VMEM((B * D,), jnp.float32),
            pltpu.SMEM((B + 1,), jnp.int32),
            pltpu.SemaphoreType.DMA,
            pltpu.SemaphoreType.DMA,
            pltpu.SemaphoreType.DMA,
        ],
        compiler_params=pltpu.CompilerParams(needs_layout_passes=False),
    )
    def k(x_hbm, nv_hbm, part_hbm, nv_v, b0, b1, b2, acc_v, bnd_s,
          s0, s1, s2):
        wid = lax.axis_index("s") * 2 + lax.axis_index("c")

        pltpu.sync_copy(nv_hbm, nv_v)
        nv_vec = nv_v[...]
        lanes = lax.iota(jnp.int32, LANES)

        incl = nv_vec
        for sh in (1, 2, 4, 8):
            shifted = incl.at[jnp.maximum(lanes - sh, 0)].get(
                mode="promise_in_bounds")
            incl = incl + jnp.where(lanes >= sh, shifted, 0)
        starts_vec = incl - nv_vec
        for s in range(B):
            bnd_s[s] = jnp.sum(jnp.where(lanes == s, starts_vec, 0))
        bnd_s[B] = jnp.sum(jnp.where(lanes == B - 1, incl, 0))

        zero = jnp.zeros((LANES,), jnp.float32)
        for i in range(B * D // LANES):
            acc_v[pl.ds(i * LANES, LANES)] = zero

        row_lo = wid * RPW
        bufs = [b0, b1, b2]
        sems = [s0, s1, s2]

        def start_chunk(kk, which):
            base = row_lo + kk * CH
            return pltpu.async_copy(
                x_hbm.at[pl.ds(base * D, CH * D)], bufs[which], sems[which])

        def consume(kk, buf_v):
            base = row_lo + kk * CH

            def seg_body(s, carry):
                a = jnp.clip(bnd_s[s] - base, 0, CH)
                b = jnp.clip(bnd_s[s + 1] - base, 0, CH)

                @pl.when(b > a)
                def _():
                    zcarry = tuple(zero for _ in range(G))

                    @plsc.parallel_loop(a, b, unroll=2, carry=zcarry)
                    def row_body(rr, vs):
                        return tuple(
                            vs[g] + buf_v[pl.ds(rr * D + g * LANES, LANES)]
                            for g in range(G))

                    for g in range(G):
                        off = s * D + g * LANES
                        acc_v[pl.ds(off, LANES)] += row_body[g]

                return carry

            lax.fori_loop(0, B, seg_body, 0)

        NBUF = 3
        for kk in range(NBUF - 1):
            start_chunk(kk, kk)
        for kk in range(NCH):
            w = kk % NBUF
            pltpu.make_async_copy(
                x_hbm.at[pl.ds(0, CH * D)], bufs[w], sems[w]).wait()
            if kk + NBUF - 1 < NCH:
                start_chunk(kk + NBUF - 1, (kk + NBUF - 1) % NBUF)
            consume(kk, bufs[w])
        pltpu.sync_copy(acc_v, part_hbm.at[pl.ds(wid * B * D, B * D)])

    return k(x_flat, nv)


def _tc_segsum(x2d, starts, ends):
    nblk = (TOTAL - SC_ROWS) // RB

    def body(x_ref, s_ref, e_ref, o_ref):
        k = pl.program_id(0)
        rows = (SC_ROWS + k * RB
                + jax.lax.broadcasted_iota(jnp.int32, (B, RB), 1))
        mask = ((rows >= s_ref[...]) & (rows < e_ref[...])).astype(jnp.float32)
        part = jnp.dot(mask, x_ref[...], preferred_element_type=jnp.float32)

        @pl.when(k == 0)
        def _():
            o_ref[...] = jnp.zeros_like(o_ref)

        o_ref[...] += part

    return pl.pallas_call(
        body,
        grid=(nblk,),
        in_specs=[
            pl.BlockSpec((RB, D), lambda k: (SC_ROWS // RB + k, 0)),
            pl.BlockSpec((B, 1), lambda k: (0, 0)),
            pl.BlockSpec((B, 1), lambda k: (0, 0)),
        ],
        out_specs=pl.BlockSpec((B, D), lambda k: (0, 0)),
        out_shape=jax.ShapeDtypeStruct((B, D), jnp.float32),
    )(x2d, starts, ends)


def _tc_combine(partials, tc_part, nv):
    def body(p_ref, t_ref, c_ref, o_ref):
        acc = t_ref[...]
        for i in range(NW):
            acc = acc + p_ref[i]
        counts = jnp.maximum(c_ref[...].astype(jnp.float32), 1.0)
        o_ref[...] = acc / counts

    return pl.pallas_call(
        body,
        out_shape=jax.ShapeDtypeStruct((B, D), jnp.float32),
    )(partials, tc_part, nv)


def kernel(inputs, nv_in):
    x_flat = inputs.reshape(-1)
    part = _sc_partial_sums(x_flat, nv_in)
    ends = jnp.cumsum(nv_in).astype(jnp.int32).reshape(B, 1)
    starts = ends - nv_in.reshape(B, 1)
    tc_part = _tc_segsum(inputs, starts, ends)
    partials = part.reshape(NW, B, D)
    return _tc_combine(partials, tc_part, nv_in.reshape(B, 1))


# speedup vs baseline: 1.1110x; 1.0148x over previous
import functools

import jax
import jax.numpy as jnp
from jax import lax
from jax.experimental import pallas as pl
from jax.experimental.pallas import tpu as pltpu
from jax.experimental.pallas import tpu_sc as plsc

B = 16
D = 128
TOTAL = 32768
SC_ROWS = TOTAL // 2
NW = 32
RPW = SC_ROWS // NW
CH = 128
NCH = RPW // CH
LANES = 16
G = D // LANES
RB = 2048


def _sc_partial_sums(x_flat, nv):
    mesh = plsc.VectorSubcoreMesh(core_axis_name="c", subcore_axis_name="s")

    @functools.partial(
        pl.kernel,
        mesh=mesh,
        out_type=jax.ShapeDtypeStruct((NW * B * D,), jnp.float32),
        scratch_types=[
            pltpu.VMEM((B,), jnp.int32),
            pltpu.VMEM((CH * D,), jnp.float32),
            pltpu.VMEM((CH * D,), jnp.float32),
            pltpu.VMEM((CH * D,), jnp.float32),
            pltpu.VMEM((B * D,), jnp.float32),
            pltpu.SMEM((B + 1,), jnp.int32),
            pltpu.SemaphoreType.DMA,
            pltpu.SemaphoreType.DMA,
            pltpu.SemaphoreType.DMA,
        ],
        compiler_params=pltpu.CompilerParams(needs_layout_passes=False),
    )
    def k(x_hbm, nv_hbm, part_hbm, nv_v, b0, b1, b2, acc_v, bnd_s,
          s0, s1, s2):
        wid = lax.axis_index("s") * 2 + lax.axis_index("c")

        pltpu.sync_copy(nv_hbm, nv_v)
        nv_vec = nv_v[...]
        lanes = lax.iota(jnp.int32, LANES)

        incl = nv_vec
        for sh in (1, 2, 4, 8):
            shifted = incl.at[jnp.maximum(lanes - sh, 0)].get(
                mode="promise_in_bounds")
            incl = incl + jnp.where(lanes >= sh, shifted, 0)
        starts_vec = incl - nv_vec
        for s in range(B):
            bnd_s[s] = jnp.sum(jnp.where(lanes == s, starts_vec, 0))
        bnd_s[B] = jnp.sum(jnp.where(lanes == B - 1, incl, 0))

        zero = jnp.zeros((LANES,), jnp.float32)
        for i in range(B * D // LANES):
            acc_v[pl.ds(i * LANES, LANES)] = zero

        row_lo = wid * RPW
        bufs = [b0, b1, b2]
        sems = [s0, s1, s2]

        def start_chunk(kk, which):
            base = row_lo + kk * CH
            return pltpu.async_copy(
                x_hbm.at[pl.ds(base * D, CH * D)], bufs[which], sems[which])

        def consume(kk, buf_v):
            base = row_lo + kk * CH

            def seg_body(s, carry):
                a = jnp.clip(bnd_s[s] - base, 0, CH)
                b = jnp.clip(bnd_s[s + 1] - base, 0, CH)

                @pl.when(b > a)
                def _():
                    zcarry = tuple(zero for _ in range(G))

                    @plsc.parallel_loop(a, b, unroll=2, carry=zcarry)
                    def row_body(rr, vs):
                        return tuple(
                            vs[g] + buf_v[pl.ds(rr * D + g * LANES, LANES)]
                            for g in range(G))

                    for g in range(G):
                        off = s * D + g * LANES
                        acc_v[pl.ds(off, LANES)] += row_body[g]

                return carry

            lax.fori_loop(0, B, seg_body, 0)

        NBUF = 3
        for kk in range(NBUF - 1):
            start_chunk(kk, kk)
        for kk in range(NCH):
            w = kk % NBUF
            pltpu.make_async_copy(
                x_hbm.at[pl.ds(0, CH * D)], bufs[w], sems[w]).wait()
            if kk + NBUF - 1 < NCH:
                start_chunk(kk + NBUF - 1, (kk + NBUF - 1) % NBUF)
            consume(kk, bufs[w])
        pltpu.sync_copy(acc_v, part_hbm.at[pl.ds(wid * B * D, B * D)])

    return k(x_flat, nv)


def _tc_segsum(x2d, starts, ends):
    nblk = (TOTAL - SC_ROWS) // RB

    def body(x_ref, s_ref, e_ref, o_ref):
        k = pl.program_id(0)
        rows = (SC_ROWS + k * RB
                + jax.lax.broadcasted_iota(jnp.int32, (B, RB), 1))
        mask = ((rows >= s_ref[...]) & (rows < e_ref[...])).astype(jnp.float32)
        part = jnp.dot(mask, x_ref[...], preferred_element_type=jnp.float32,
                       precision=jax.lax.Precision.HIGHEST)

        @pl.when(k == 0)
        def _():
            o_ref[...] = jnp.zeros_like(o_ref)

        o_ref[...] += part

    return pl.pallas_call(
        body,
        grid=(nblk,),
        in_specs=[
            pl.BlockSpec((RB, D), lambda k: (SC_ROWS // RB + k, 0)),
            pl.BlockSpec((B, 1), lambda k: (0, 0)),
            pl.BlockSpec((B, 1), lambda k: (0, 0)),
        ],
        out_specs=pl.BlockSpec((B, D), lambda k: (0, 0)),
        out_shape=jax.ShapeDtypeStruct((B, D), jnp.float32),
    )(x2d, starts, ends)


def _tc_combine(partials, tc_part, nv):
    def body(p_ref, t_ref, c_ref, o_ref):
        acc = t_ref[...]
        for i in range(NW):
            acc = acc + p_ref[i]
        counts = jnp.maximum(c_ref[...].astype(jnp.float32), 1.0)
        o_ref[...] = acc / counts

    return pl.pallas_call(
        body,
        out_shape=jax.ShapeDtypeStruct((B, D), jnp.float32),
    )(partials, tc_part, nv)


def kernel(inputs, nv_in):
    x_flat = inputs.reshape(-1)
    part = _sc_partial_sums(x_flat, nv_in)
    ends = jnp.cumsum(nv_in).astype(jnp.int32).reshape(B, 1)
    starts = ends - nv_in.reshape(B, 1)
    tc_part = _tc_segsum(inputs, starts, ends)
    partials = part.reshape(NW, B, D)
    return _tc_combine(partials, tc_part, nv_in.reshape(B, 1))
